# baseline (speedup 1.0000x reference)
import jax
import jax.numpy as jnp
import numpy as np
from jax.experimental import pallas as pl
from jax.experimental.pallas import tpu as pltpu

_H = 128
_N = _H * _H
_PAD = 640
_RB = 1024
_NBLK = _N // _RB

_ii, _jj = np.meshgrid(np.arange(_H), np.arange(_H), indexing="ij")
_iif, _jjf = _ii.ravel(), _jj.ravel()
_srcs = []
for _di, _dj in [(-1, -1), (-1, 0), (-1, 1), (0, -1), (0, 0), (0, 1),
                 (-1, -1), (-1, 0), (-1, 1)]:
    _srcs.append(((_iif + _di) % _H) * _H + (_jjf + _dj) % _H)
_src = np.concatenate(_srcs)
_dst = np.repeat((_iif * 4 + _jjf)[:, None], 9, axis=1).ravel()
_deg = np.ones(_N, dtype=np.float64)
np.add.at(_deg, _dst, 1.0)
_dinv = 1.0 / np.sqrt(_deg)

_P = np.zeros((_PAD, _N), dtype=np.float64)
np.add.at(_P, (_dst, _src), _dinv[_dst] * _dinv[_src])
_P16 = _P.astype(jnp.bfloat16)
_DINV2 = (_dinv * _dinv).astype(np.float32).reshape(_N, 1)

_F32 = jnp.float32


def _mega_body(x_ref, p_ref, d2_ref, w1_ref, wstk_ref, bstk_ref,
               o_ref, h_ref, agg_ref):
    s = pl.program_id(0)
    l = s // _NBLK
    i = s % _NBLK
    r = (i + 1) % _NBLK
    off = r * _RB

    def phase(xfull, xslice, w, relu, store, oneshot):
        xb = xslice()
        if oneshot:
            @pl.when(i == 0)
            def _ex():
                t = jnp.dot(p_ref[:], xfull(), preferred_element_type=_F32)
                agg_ref[:] = jnp.dot(t.astype(jnp.bfloat16), w,
                                     preferred_element_type=_F32)
        else:
            @pl.when(i == 0)
            def _zero():
                agg_ref[:] = jnp.zeros_like(agg_ref)

            agg_ref[:] += jnp.dot(p_ref[:, pl.ds(off, _RB)], xb,
                                  preferred_element_type=_F32)

        raw = (d2_ref[pl.ds(off, _RB), :] *
               jnp.dot(xb, w, preferred_element_type=_F32)
               ) + bstk_ref[0]
        store(jnp.maximum(raw, 0.0) if relu else raw, False)

        @pl.when(i == _NBLK - 1)
        def _top():
            if oneshot:
                ex = agg_ref[:]
            else:
                ex = jnp.dot(agg_ref[:].astype(jnp.bfloat16), w,
                             preferred_element_type=_F32)
            top = jax.lax.slice(raw, (0, 0), (_PAD, 512)) + ex
            store(jnp.maximum(top, 0.0) if relu else top, True)

    def store_h(v, is_top):
        if is_top:
            h_ref[0:_PAD, :] = v.astype(jnp.bfloat16)
        else:
            h_ref[pl.ds(off, _RB), :] = v.astype(jnp.bfloat16)

    def store_o(v, is_top):
        if is_top:
            o_ref[0:_PAD, :] = v
        else:
            o_ref[:] = v

    @pl.when(l == 0)
    def _l0():
        phase(lambda: x_ref[:], lambda: x_ref[pl.ds(off, _RB), :],
              w1_ref[:], True, store_h, oneshot=True)

    @pl.when(l == 1)
    def _l1():
        phase(lambda: h_ref[:], lambda: h_ref[pl.ds(off, _RB), :],
              wstk_ref[0], True, store_h, oneshot=False)

    @pl.when(l == 2)
    def _l2():
        phase(lambda: h_ref[:], lambda: h_ref[pl.ds(off, _RB), :],
              wstk_ref[0], False, store_o, oneshot=False)


def _gcn3(xv, W1, b1, W2, b2, W3, b3):
    wstk = jnp.stack([W2.astype(jnp.bfloat16), W3.astype(jnp.bfloat16)])
    bstk = jnp.stack([b1, b2, b3]).reshape(3, 1, 512)
    const = lambda s: (0, 0)
    return pl.pallas_call(
        _mega_body,
        grid=(3 * _NBLK,),
        in_specs=[
            pl.BlockSpec((_N, 4), const),
            pl.BlockSpec((_PAD, _N), const),
            pl.BlockSpec((_N, 1), const),
            pl.BlockSpec((4, 512), const),
            pl.BlockSpec((1, 512, 512),
                         lambda s: (jnp.maximum(s // _NBLK - 1, 0), 0, 0)),
            pl.BlockSpec((1, 1, 512), lambda s: (s // _NBLK, 0, 0)),
        ],
        out_specs=pl.BlockSpec(
            (_RB, 512),
            lambda s: (jnp.where(s < 2 * _NBLK, 1, (s % _NBLK + 1) % _NBLK),
                       0)),
        out_shape=jax.ShapeDtypeStruct((_N, 512), jnp.float32),
        scratch_shapes=[pltpu.VMEM((_N, 512), jnp.bfloat16),
                        pltpu.VMEM((_PAD, 512), jnp.float32)],
    )(xv, _P16, _DINV2, W1.astype(jnp.bfloat16), wstk, bstk)


def kernel(x, W1, b1, W2, b2, W3, b3):
    xv = x.reshape(_N, 4).astype(jnp.bfloat16)
    h3 = _gcn3(xv, W1, b1, W2, b2, W3, b3)
    return h3.reshape(1, _N, 512)
